# baseline (speedup 1.0000x reference)
import functools

import jax
import jax.numpy as jnp
from jax.experimental import pallas as pl

B = 1024
L = 50
D = 64
MP = 9
NA = 16
C = 7

EV_BLK = 3200
B_BLK = EV_BLK // L


def _prep_kernel(card_tab_ref, src_tab_ref, hero_tab_ref, actpos_tab_ref,
                 np_tab_ref, Ws_ref, bs_ref, Wb_ref, bb_ref, Wa_ref, ba_ref,
                 Wc_ref, bc_ref, gamma_ref, beta_ref,
                 card_proj_ref, wctx_ref, bias_ref, beta_flat_ref, g7_ref,
                 bd_ref):
    Wc = Wc_ref[...]
    wc_card = Wc[0:D, :]
    wc_hero = Wc[D:2 * D, :]
    wc_act = Wc[2 * D:3 * D, :]
    wc_np = Wc[3 * D:4 * D, :]
    wc_s = Wc[4 * D:5 * D, :]
    wc_b = Wc[5 * D:6 * D, :]
    wc_a = Wc[6 * D:7 * D, :]
    f32 = jnp.float32
    dot = functools.partial(jnp.dot, preferred_element_type=f32,
                            precision=jax.lax.Precision.HIGHEST)
    card_proj = dot(card_tab_ref[...], wc_card)
    card_proj_ref[...] = card_proj - jnp.mean(card_proj, axis=1, keepdims=True)
    wctx = jnp.concatenate([
        dot(Ws_ref[...], wc_s),
        dot(Wb_ref[...], wc_b),
        dot(Wa_ref[...], wc_a),
        dot(hero_tab_ref[...], wc_hero),
        dot(actpos_tab_ref[...], wc_act),
        dot(np_tab_ref[...], wc_np),
    ], axis=0)
    wctx_ref[...] = wctx - jnp.mean(wctx, axis=1, keepdims=True)
    bias = (bc_ref[...] + dot(bs_ref[...], wc_s)
            + dot(bb_ref[...], wc_b) + dot(ba_ref[...], wc_a))
    bias_ref[...] = bias - jnp.mean(bias, axis=1, keepdims=True)
    src = src_tab_ref[...]
    beta7 = beta_ref[...] + jnp.concatenate(
        [jnp.broadcast_to(src[0:1, :], (5, D)),
         jnp.broadcast_to(src[1:2, :], (2, D))], axis=0)
    beta_flat_ref[...] = jnp.concatenate(
        [beta7[c:c + 1, :] for c in range(C)], axis=1)
    z = jnp.zeros((1, D), f32)
    g_rows = []
    for c in range(C):
        g_rows.append(jnp.concatenate(
            [z] * c + [gamma_ref[...]] + [z] * (C - 1 - c), axis=1))
    g7_ref[...] = jnp.concatenate(g_rows, axis=0)
    bd_ref[...] = (jax.lax.broadcasted_iota(jnp.int32, (C * D, C), 0) // D
                   == jax.lax.broadcasted_iota(jnp.int32, (C * D, C), 1)
                   ).astype(f32)


_ICOL_HERO = C
_ICOL_ACT = C + 1
_ICOL_NPL = C + 2
_ICOL_SEQ = C + 3
_N_ICOLS = C + 4


def _main_kernel(ints_ref, scalars_ref, bets_ref, action_ref,
                 card_proj_ref, wctx_ref, bias_ref,
                 beta_flat_ref, g7_ref, bd_ref, out_ref):
    f32 = jnp.float32
    i32 = jnp.int32
    N = EV_BLK
    dot = functools.partial(jnp.dot, preferred_element_type=f32)
    ints = ints_ref[...].reshape(N, _N_ICOLS)
    ioh = jax.lax.broadcasted_iota(i32, (N, MP), 1)
    ion = jax.lax.broadcasted_iota(i32, (N, MP + 1), 1)
    feats = jnp.concatenate([
        scalars_ref[...].reshape(N, 2),
        bets_ref[...].reshape(N, MP),
        action_ref[...].reshape(N, NA),
        (ints[:, _ICOL_HERO:_ICOL_HERO + 1] == ioh).astype(f32),
        (ints[:, _ICOL_ACT:_ICOL_ACT + 1] == ioh).astype(f32),
        (ints[:, _ICOL_NPL:_ICOL_NPL + 1] == ion).astype(f32),
    ], axis=1)
    ctx = dot(feats, wctx_ref[...]) + bias_ref[...]
    lpos = jnp.remainder(
        jax.lax.broadcasted_iota(i32, (N, 1), 0), L)
    m = (lpos < ints[:, _ICOL_SEQ:_ICOL_SEQ + 1]).astype(f32)
    ioc = jax.lax.broadcasted_iota(i32, (N, 53), 1)
    card_proj = card_proj_ref[...]
    xc_all = jnp.concatenate([
        dot((ints[:, c:c + 1] == ioc).astype(f32), card_proj) + ctx
        for c in range(C)
    ], axis=1)
    s7 = dot(xc_all * xc_all, bd_ref[...])
    rm7 = jax.lax.rsqrt(s7 * (1.0 / D) + 1e-5)
    rm7 = rm7 * dot(m, jnp.ones((1, C), f32))
    a_all = dot(rm7, g7_ref[...])
    b_all = dot(m, beta_flat_ref[...])
    out_ref[...] = xc_all * a_all + b_all


def _mask_kernel(seq_ref, mask_ref):
    i350 = jax.lax.broadcasted_iota(jnp.int32, (B_BLK, L * C), 1)
    mask_ref[...] = (i350 // C < seq_ref[...]).astype(jnp.float32)


def kernel(card_ids, hero_pos, acting_pos, num_players, scalars, bets, action,
           seq_lengths, card_tab, src_tab, hero_tab, actpos_tab, np_tab,
           Ws, bs, Wb, bb, Wa, ba, Wc, bc, gamma, beta):
    f32 = jnp.float32
    i32 = jnp.int32
    card_proj, wctx, bias, beta_flat, g7, bd = pl.pallas_call(
        _prep_kernel,
        out_shape=(
            jax.ShapeDtypeStruct((53, D), f32),
            jax.ShapeDtypeStruct((55, D), f32),
            jax.ShapeDtypeStruct((1, D), f32),
            jax.ShapeDtypeStruct((1, C * D), f32),
            jax.ShapeDtypeStruct((C, C * D), f32),
            jax.ShapeDtypeStruct((C * D, C), f32),
        ),
    )(card_tab, src_tab, hero_tab, actpos_tab, np_tab,
      Ws, bs.reshape(1, D), Wb, bb.reshape(1, D), Wa, ba.reshape(1, D),
      Wc, bc.reshape(1, D), gamma.reshape(1, D), beta.reshape(1, D))

    BL = B * L
    ints = jnp.concatenate([
        card_ids.astype(i32),
        hero_pos.astype(i32)[:, :, None],
        acting_pos.astype(i32)[:, :, None],
        num_players.astype(i32)[:, :, None],
        jnp.broadcast_to(seq_lengths.astype(i32)[:, None, None], (B, L, 1)),
    ], axis=2)

    grid = (BL // EV_BLK,)
    const2 = lambda shape: pl.BlockSpec(shape, lambda i: (0, 0))
    emb = pl.pallas_call(
        _main_kernel,
        grid=grid,
        in_specs=[
            pl.BlockSpec((B_BLK, L, _N_ICOLS), lambda i: (i, 0, 0)),
            pl.BlockSpec((B_BLK, L, 2), lambda i: (i, 0, 0)),
            pl.BlockSpec((B_BLK, L, MP), lambda i: (i, 0, 0)),
            pl.BlockSpec((B_BLK, L, NA), lambda i: (i, 0, 0)),
            const2((53, D)), const2((55, D)), const2((1, D)),
            const2((1, C * D)), const2((C, C * D)), const2((C * D, C)),
        ],
        out_specs=pl.BlockSpec((EV_BLK, C * D), lambda i: (i, 0)),
        out_shape=jax.ShapeDtypeStruct((BL, C * D), f32),
    )(ints, scalars, bets, action, card_proj, wctx, bias, beta_flat, g7, bd)

    mask = pl.pallas_call(
        _mask_kernel,
        grid=grid,
        in_specs=[pl.BlockSpec((B_BLK, 1), lambda i: (i, 0))],
        out_specs=pl.BlockSpec((B_BLK, L * C), lambda i: (i, 0)),
        out_shape=jax.ShapeDtypeStruct((B, L * C), f32),
    )(seq_lengths.astype(i32).reshape(B, 1))
    return emb.reshape(B, L * C, D), mask


# speedup vs baseline: 1.2843x; 1.2843x over previous
import functools

import jax
import jax.numpy as jnp
from jax.experimental import pallas as pl

B = 1024
L = 50
D = 64
MP = 9
NA = 16
C = 7

EV_BLK = 3200
B_BLK = EV_BLK // L


def _prep_kernel(card_tab_ref, src_tab_ref, hero_tab_ref, actpos_tab_ref,
                 np_tab_ref, Ws_ref, bs_ref, Wb_ref, bb_ref, Wa_ref, ba_ref,
                 Wc_ref, bc_ref, gamma_ref, beta_ref,
                 card_proj_ref, wctx_ref, bias_ref, beta_flat_ref, g7_ref,
                 bd_ref):
    Wc = Wc_ref[...]
    wc_card = Wc[0:D, :]
    wc_hero = Wc[D:2 * D, :]
    wc_act = Wc[2 * D:3 * D, :]
    wc_np = Wc[3 * D:4 * D, :]
    wc_s = Wc[4 * D:5 * D, :]
    wc_b = Wc[5 * D:6 * D, :]
    wc_a = Wc[6 * D:7 * D, :]
    f32 = jnp.float32
    dot = functools.partial(jnp.dot, preferred_element_type=f32,
                            precision=jax.lax.Precision.HIGHEST)
    card_proj = dot(card_tab_ref[...], wc_card)
    card_proj_ref[...] = card_proj - jnp.mean(card_proj, axis=1, keepdims=True)
    wctx = jnp.concatenate([
        dot(Ws_ref[...], wc_s),
        dot(Wb_ref[...], wc_b),
        dot(Wa_ref[...], wc_a),
        dot(hero_tab_ref[...], wc_hero),
        dot(actpos_tab_ref[...], wc_act),
        dot(np_tab_ref[...], wc_np),
    ], axis=0)
    wctx_ref[...] = wctx - jnp.mean(wctx, axis=1, keepdims=True)
    bias = (bc_ref[...] + dot(bs_ref[...], wc_s)
            + dot(bb_ref[...], wc_b) + dot(ba_ref[...], wc_a))
    bias_ref[...] = bias - jnp.mean(bias, axis=1, keepdims=True)
    src = src_tab_ref[...]
    beta7 = beta_ref[...] + jnp.concatenate(
        [jnp.broadcast_to(src[0:1, :], (5, D)),
         jnp.broadcast_to(src[1:2, :], (2, D))], axis=0)
    beta_flat_ref[...] = jnp.concatenate(
        [beta7[c:c + 1, :] for c in range(C)], axis=1)
    z = jnp.zeros((1, D), f32)
    g_rows = []
    for c in range(C):
        g_rows.append(jnp.concatenate(
            [z] * c + [gamma_ref[...]] + [z] * (C - 1 - c), axis=1))
    g7_ref[...] = jnp.concatenate(g_rows, axis=0)
    bd_ref[...] = (jax.lax.broadcasted_iota(jnp.int32, (C * D, C), 0) // D
                   == jax.lax.broadcasted_iota(jnp.int32, (C * D, C), 1)
                   ).astype(f32)


_COL_DENSE_END = 2 + MP + NA
_COL_CARDS = _COL_DENSE_END
_COL_HERO = _COL_CARDS + C
_COL_ACT = _COL_HERO + 1
_COL_NPL = _COL_ACT + 1
_COL_SEQ = _COL_NPL + 1
_N_COLS = _COL_SEQ + 1


def _main_kernel(packed_ref, card_proj_ref, wctx_ref, bias_ref,
                 beta_flat_ref, g7_ref, bd_ref, out_ref):
    f32 = jnp.float32
    N = EV_BLK
    dot = functools.partial(jnp.dot, preferred_element_type=f32)
    p = packed_ref[...]
    iohf = jax.lax.broadcasted_iota(jnp.int32, (N, MP), 1).astype(f32)
    ionf = jax.lax.broadcasted_iota(jnp.int32, (N, MP + 1), 1).astype(f32)
    feats = jnp.concatenate([
        p[:, 0:_COL_DENSE_END],
        (p[:, _COL_HERO:_COL_HERO + 1] == iohf).astype(f32),
        (p[:, _COL_ACT:_COL_ACT + 1] == iohf).astype(f32),
        (p[:, _COL_NPL:_COL_NPL + 1] == ionf).astype(f32),
    ], axis=1)
    ctx = dot(feats, wctx_ref[...]) + bias_ref[...]
    lpos = jnp.remainder(
        jax.lax.broadcasted_iota(jnp.int32, (N, 1), 0), L).astype(f32)
    m = (lpos < p[:, _COL_SEQ:_COL_SEQ + 1]).astype(f32)
    iocf = jax.lax.broadcasted_iota(jnp.int32, (N, 53), 1).astype(f32)
    card_proj = card_proj_ref[...]
    xc_all = jnp.concatenate([
        dot((p[:, _COL_CARDS + c:_COL_CARDS + c + 1] == iocf).astype(f32),
            card_proj) + ctx
        for c in range(C)
    ], axis=1)
    s7 = dot(xc_all * xc_all, bd_ref[...])
    rm7 = jax.lax.rsqrt(s7 * (1.0 / D) + 1e-5)
    rm7 = rm7 * dot(m, jnp.ones((1, C), f32))
    a_all = dot(rm7, g7_ref[...])
    b_all = dot(m, beta_flat_ref[...])
    out_ref[...] = xc_all * a_all + b_all


def _mask_kernel(seq_ref, mask_ref):
    i350 = jax.lax.broadcasted_iota(jnp.int32, (B_BLK, L * C), 1)
    mask_ref[...] = (i350 // C < seq_ref[...]).astype(jnp.float32)


def kernel(card_ids, hero_pos, acting_pos, num_players, scalars, bets, action,
           seq_lengths, card_tab, src_tab, hero_tab, actpos_tab, np_tab,
           Ws, bs, Wb, bb, Wa, ba, Wc, bc, gamma, beta):
    f32 = jnp.float32
    i32 = jnp.int32
    card_proj, wctx, bias, beta_flat, g7, bd = pl.pallas_call(
        _prep_kernel,
        out_shape=(
            jax.ShapeDtypeStruct((53, D), f32),
            jax.ShapeDtypeStruct((55, D), f32),
            jax.ShapeDtypeStruct((1, D), f32),
            jax.ShapeDtypeStruct((1, C * D), f32),
            jax.ShapeDtypeStruct((C, C * D), f32),
            jax.ShapeDtypeStruct((C * D, C), f32),
        ),
    )(card_tab, src_tab, hero_tab, actpos_tab, np_tab,
      Ws, bs.reshape(1, D), Wb, bb.reshape(1, D), Wa, ba.reshape(1, D),
      Wc, bc.reshape(1, D), gamma.reshape(1, D), beta.reshape(1, D))

    def _pack(b0, b1):
        nb = b1 - b0
        seqf = jnp.broadcast_to(
            seq_lengths[b0:b1].astype(f32).reshape(nb, 1, 1), (nb, L, 1))
        return jnp.concatenate([
            scalars[b0:b1], bets[b0:b1], action[b0:b1],
            card_ids[b0:b1].astype(f32),
            hero_pos[b0:b1].astype(f32)[:, :, None],
            acting_pos[b0:b1].astype(f32)[:, :, None],
            num_players[b0:b1].astype(f32)[:, :, None],
            seqf,
        ], axis=2).reshape(nb * L, _N_COLS)

    const2 = lambda shape: pl.BlockSpec(shape, lambda i: (0, 0))

    def _run(packed):
        bl = packed.shape[0]
        return pl.pallas_call(
            _main_kernel,
            grid=(bl // EV_BLK,),
            in_specs=[
                pl.BlockSpec((EV_BLK, _N_COLS), lambda i: (i, 0)),
                const2((53, D)), const2((55, D)), const2((1, D)),
                const2((1, C * D)), const2((C, C * D)), const2((C * D, C)),
            ],
            out_specs=pl.BlockSpec((EV_BLK, C * D), lambda i: (i, 0)),
            out_shape=jax.ShapeDtypeStruct((bl, C * D), f32),
        )(packed, card_proj, wctx, bias, beta_flat, g7, bd)

    half = B // 2
    emb_a = _run(_pack(0, half))
    emb_b = _run(_pack(half, B))
    emb = jnp.concatenate([
        emb_a.reshape(half, L * C, D),
        emb_b.reshape(half, L * C, D),
    ], axis=0)

    mask = pl.pallas_call(
        _mask_kernel,
        grid=(B * L // EV_BLK,),
        in_specs=[pl.BlockSpec((B_BLK, 1), lambda i: (i, 0))],
        out_specs=pl.BlockSpec((B_BLK, L * C), lambda i: (i, 0)),
        out_shape=jax.ShapeDtypeStruct((B, L * C), f32),
    )(seq_lengths.astype(i32).reshape(B, 1))
    return emb, mask


# speedup vs baseline: 1.2975x; 1.0102x over previous
import functools

import jax
import jax.numpy as jnp
from jax.experimental import pallas as pl

B = 1024
L = 50
D = 64
MP = 9
NA = 16
C = 7

EV_BLK = 1600
B_BLK = EV_BLK // L


def _prep_kernel(card_tab_ref, src_tab_ref, hero_tab_ref, actpos_tab_ref,
                 np_tab_ref, Ws_ref, bs_ref, Wb_ref, bb_ref, Wa_ref, ba_ref,
                 Wc_ref, bc_ref, gamma_ref, beta_ref,
                 card_proj_ref, wctx_ref, bias_ref, beta_flat_ref, g7_ref,
                 bd_ref):
    Wc = Wc_ref[...]
    wc_card = Wc[0:D, :]
    wc_hero = Wc[D:2 * D, :]
    wc_act = Wc[2 * D:3 * D, :]
    wc_np = Wc[3 * D:4 * D, :]
    wc_s = Wc[4 * D:5 * D, :]
    wc_b = Wc[5 * D:6 * D, :]
    wc_a = Wc[6 * D:7 * D, :]
    f32 = jnp.float32
    dot = functools.partial(jnp.dot, preferred_element_type=f32,
                            precision=jax.lax.Precision.HIGHEST)
    card_proj = dot(card_tab_ref[...], wc_card)
    card_proj_ref[...] = card_proj - jnp.mean(card_proj, axis=1, keepdims=True)
    wctx = jnp.concatenate([
        dot(Ws_ref[...], wc_s),
        dot(Wb_ref[...], wc_b),
        dot(Wa_ref[...], wc_a),
        dot(hero_tab_ref[...], wc_hero),
        dot(actpos_tab_ref[...], wc_act),
        dot(np_tab_ref[...], wc_np),
    ], axis=0)
    wctx_ref[...] = wctx - jnp.mean(wctx, axis=1, keepdims=True)
    bias = (bc_ref[...] + dot(bs_ref[...], wc_s)
            + dot(bb_ref[...], wc_b) + dot(ba_ref[...], wc_a))
    bias_ref[...] = bias - jnp.mean(bias, axis=1, keepdims=True)
    src = src_tab_ref[...]
    beta7 = beta_ref[...] + jnp.concatenate(
        [jnp.broadcast_to(src[0:1, :], (5, D)),
         jnp.broadcast_to(src[1:2, :], (2, D))], axis=0)
    beta_flat_ref[...] = jnp.concatenate(
        [beta7[c:c + 1, :] for c in range(C)], axis=1)
    z = jnp.zeros((1, D), f32)
    g_rows = []
    for c in range(C):
        g_rows.append(jnp.concatenate(
            [z] * c + [gamma_ref[...]] + [z] * (C - 1 - c), axis=1))
    g7_ref[...] = jnp.concatenate(g_rows, axis=0)
    bd_ref[...] = (jax.lax.broadcasted_iota(jnp.int32, (C * D, C), 0) // D
                   == jax.lax.broadcasted_iota(jnp.int32, (C * D, C), 1)
                   ).astype(f32)


_COL_DENSE_END = 2 + MP + NA
_COL_CARDS = _COL_DENSE_END
_COL_HERO = _COL_CARDS + C
_COL_ACT = _COL_HERO + 1
_COL_NPL = _COL_ACT + 1
_COL_SEQ = _COL_NPL + 1
_N_COLS = _COL_SEQ + 1


def _main_kernel(packed_ref, card_proj_ref, wctx_ref, bias_ref,
                 beta_flat_ref, g7_ref, bd_ref, out_ref):
    f32 = jnp.float32
    N = EV_BLK
    dot = functools.partial(jnp.dot, preferred_element_type=f32)
    p = packed_ref[...]
    iohf = jax.lax.broadcasted_iota(jnp.int32, (N, MP), 1).astype(f32)
    ionf = jax.lax.broadcasted_iota(jnp.int32, (N, MP + 1), 1).astype(f32)
    feats = jnp.concatenate([
        p[:, 0:_COL_DENSE_END],
        (p[:, _COL_HERO:_COL_HERO + 1] == iohf).astype(f32),
        (p[:, _COL_ACT:_COL_ACT + 1] == iohf).astype(f32),
        (p[:, _COL_NPL:_COL_NPL + 1] == ionf).astype(f32),
    ], axis=1)
    ctx = dot(feats, wctx_ref[...]) + bias_ref[...]
    lpos = jnp.remainder(
        jax.lax.broadcasted_iota(jnp.int32, (N, 1), 0), L).astype(f32)
    m = (lpos < p[:, _COL_SEQ:_COL_SEQ + 1]).astype(f32)
    iocf = jax.lax.broadcasted_iota(jnp.int32, (N, 53), 1).astype(f32)
    card_proj = card_proj_ref[...]
    xc_all = jnp.concatenate([
        dot((p[:, _COL_CARDS + c:_COL_CARDS + c + 1] == iocf).astype(f32),
            card_proj) + ctx
        for c in range(C)
    ], axis=1)
    s7 = dot(xc_all * xc_all, bd_ref[...])
    rm7 = jax.lax.rsqrt(s7 * (1.0 / D) + 1e-5)
    rm7 = rm7 * dot(m, jnp.ones((1, C), f32))
    a_all = dot(rm7, g7_ref[...])
    b_all = dot(m, beta_flat_ref[...])
    out_ref[...] = xc_all * a_all + b_all


def _mask_kernel(seq_ref, mask_ref):
    i350 = jax.lax.broadcasted_iota(jnp.int32, (B_BLK, L * C), 1)
    mask_ref[...] = (i350 // C < seq_ref[...]).astype(jnp.float32)


def kernel(card_ids, hero_pos, acting_pos, num_players, scalars, bets, action,
           seq_lengths, card_tab, src_tab, hero_tab, actpos_tab, np_tab,
           Ws, bs, Wb, bb, Wa, ba, Wc, bc, gamma, beta):
    f32 = jnp.float32
    i32 = jnp.int32
    card_proj, wctx, bias, beta_flat, g7, bd = pl.pallas_call(
        _prep_kernel,
        out_shape=(
            jax.ShapeDtypeStruct((53, D), f32),
            jax.ShapeDtypeStruct((55, D), f32),
            jax.ShapeDtypeStruct((1, D), f32),
            jax.ShapeDtypeStruct((1, C * D), f32),
            jax.ShapeDtypeStruct((C, C * D), f32),
            jax.ShapeDtypeStruct((C * D, C), f32),
        ),
    )(card_tab, src_tab, hero_tab, actpos_tab, np_tab,
      Ws, bs.reshape(1, D), Wb, bb.reshape(1, D), Wa, ba.reshape(1, D),
      Wc, bc.reshape(1, D), gamma.reshape(1, D), beta.reshape(1, D))

    BL = B * L
    seqf = jnp.broadcast_to(
        seq_lengths.astype(f32).reshape(B, 1, 1), (B, L, 1))
    packed = jnp.concatenate([
        scalars, bets, action,
        card_ids.astype(f32),
        hero_pos.astype(f32)[:, :, None],
        acting_pos.astype(f32)[:, :, None],
        num_players.astype(f32)[:, :, None],
        seqf,
    ], axis=2).reshape(BL, _N_COLS)

    grid = (BL // EV_BLK,)
    const2 = lambda shape: pl.BlockSpec(shape, lambda i: (0, 0))
    emb = pl.pallas_call(
        _main_kernel,
        grid=grid,
        in_specs=[
            pl.BlockSpec((EV_BLK, _N_COLS), lambda i: (i, 0)),
            const2((53, D)), const2((55, D)), const2((1, D)),
            const2((1, C * D)), const2((C, C * D)), const2((C * D, C)),
        ],
        out_specs=pl.BlockSpec((EV_BLK, C * D), lambda i: (i, 0)),
        out_shape=jax.ShapeDtypeStruct((BL, C * D), f32),
    )(packed, card_proj, wctx, bias, beta_flat, g7, bd)

    mask = pl.pallas_call(
        _mask_kernel,
        grid=grid,
        in_specs=[pl.BlockSpec((B_BLK, 1), lambda i: (i, 0))],
        out_specs=pl.BlockSpec((B_BLK, L * C), lambda i: (i, 0)),
        out_shape=jax.ShapeDtypeStruct((B, L * C), f32),
    )(seq_lengths.astype(i32).reshape(B, 1))
    return emb.reshape(B, L * C, D), mask


# speedup vs baseline: 1.3265x; 1.0224x over previous
import functools

import jax
import jax.numpy as jnp
from jax.experimental import pallas as pl

B = 1024
L = 50
D = 64
MP = 9
NA = 16
C = 7

EV_BLK = 3200
B_BLK = EV_BLK // L


def _prep_kernel(card_tab_ref, src_tab_ref, hero_tab_ref, actpos_tab_ref,
                 np_tab_ref, Ws_ref, bs_ref, Wb_ref, bb_ref, Wa_ref, ba_ref,
                 Wc_ref, bc_ref, gamma_ref, beta_ref,
                 card_proj_ref, wctx_ref, bias_ref, beta_flat_ref, g7_ref,
                 bd_ref):
    Wc = Wc_ref[...]
    wc_card = Wc[0:D, :]
    wc_hero = Wc[D:2 * D, :]
    wc_act = Wc[2 * D:3 * D, :]
    wc_np = Wc[3 * D:4 * D, :]
    wc_s = Wc[4 * D:5 * D, :]
    wc_b = Wc[5 * D:6 * D, :]
    wc_a = Wc[6 * D:7 * D, :]
    f32 = jnp.float32
    dot = functools.partial(jnp.dot, preferred_element_type=f32,
                            precision=jax.lax.Precision.HIGHEST)
    card_proj = dot(card_tab_ref[...], wc_card)
    card_proj_ref[...] = card_proj - jnp.mean(card_proj, axis=1, keepdims=True)
    wctx = jnp.concatenate([
        dot(Ws_ref[...], wc_s),
        dot(Wb_ref[...], wc_b),
        dot(Wa_ref[...], wc_a),
        dot(hero_tab_ref[...], wc_hero),
        dot(actpos_tab_ref[...], wc_act),
        dot(np_tab_ref[...], wc_np),
    ], axis=0)
    wctx_ref[...] = wctx - jnp.mean(wctx, axis=1, keepdims=True)
    bias = (bc_ref[...] + dot(bs_ref[...], wc_s)
            + dot(bb_ref[...], wc_b) + dot(ba_ref[...], wc_a))
    bias_ref[...] = bias - jnp.mean(bias, axis=1, keepdims=True)
    src = src_tab_ref[...]
    beta7 = beta_ref[...] + jnp.concatenate(
        [jnp.broadcast_to(src[0:1, :], (5, D)),
         jnp.broadcast_to(src[1:2, :], (2, D))], axis=0)
    beta_flat_ref[...] = jnp.concatenate(
        [beta7[c:c + 1, :] for c in range(C)], axis=1)
    z = jnp.zeros((1, D), f32)
    g_rows = []
    for c in range(C):
        g_rows.append(jnp.concatenate(
            [z] * c + [gamma_ref[...]] + [z] * (C - 1 - c), axis=1))
    g7_ref[...] = jnp.concatenate(g_rows, axis=0)
    bd_ref[...] = (jax.lax.broadcasted_iota(jnp.int32, (C * D, C), 0) // D
                   == jax.lax.broadcasted_iota(jnp.int32, (C * D, C), 1)
                   ).astype(f32)


_COL_DENSE_END = 2 + MP + NA
_COL_CARDS = _COL_DENSE_END
_COL_HERO = _COL_CARDS + C
_COL_ACT = _COL_HERO + 1
_COL_NPL = _COL_ACT + 1
_COL_SEQ = _COL_NPL + 1
_N_COLS = _COL_SEQ + 1


def _main_kernel(packed_ref, seq_ref, card_proj_ref, wctx_ref, bias_ref,
                 beta_flat_ref, g7_ref, bd_ref, out_ref, mask_ref):
    f32 = jnp.float32
    N = EV_BLK
    dot = functools.partial(jnp.dot, preferred_element_type=f32)
    p = packed_ref[...]
    iohf = jax.lax.broadcasted_iota(jnp.int32, (N, MP), 1).astype(f32)
    ionf = jax.lax.broadcasted_iota(jnp.int32, (N, MP + 1), 1).astype(f32)
    feats = jnp.concatenate([
        p[:, 0:_COL_DENSE_END],
        (p[:, _COL_HERO:_COL_HERO + 1] == iohf).astype(f32),
        (p[:, _COL_ACT:_COL_ACT + 1] == iohf).astype(f32),
        (p[:, _COL_NPL:_COL_NPL + 1] == ionf).astype(f32),
    ], axis=1)
    ctx = dot(feats, wctx_ref[...]) + bias_ref[...]
    lpos = jnp.remainder(
        jax.lax.broadcasted_iota(jnp.int32, (N, 1), 0), L).astype(f32)
    m = (lpos < p[:, _COL_SEQ:_COL_SEQ + 1]).astype(f32)
    iocf = jax.lax.broadcasted_iota(jnp.int32, (N, 53), 1).astype(f32)
    card_proj = card_proj_ref[...]
    xc_all = jnp.concatenate([
        dot((p[:, _COL_CARDS + c:_COL_CARDS + c + 1] == iocf).astype(f32),
            card_proj) + ctx
        for c in range(C)
    ], axis=1)
    s7 = dot(xc_all * xc_all, bd_ref[...])
    rm7 = jax.lax.rsqrt(s7 * (1.0 / D) + 1e-5)
    rm7 = rm7 * dot(m, jnp.ones((1, C), f32))
    a_all = dot(rm7, g7_ref[...])
    b_all = dot(m, beta_flat_ref[...])
    out_ref[...] = xc_all * a_all + b_all
    i350 = jax.lax.broadcasted_iota(jnp.int32, (B_BLK, L * C), 1)
    mask_ref[...] = (i350 // C < seq_ref[...]).astype(f32)


def kernel(card_ids, hero_pos, acting_pos, num_players, scalars, bets, action,
           seq_lengths, card_tab, src_tab, hero_tab, actpos_tab, np_tab,
           Ws, bs, Wb, bb, Wa, ba, Wc, bc, gamma, beta):
    f32 = jnp.float32
    i32 = jnp.int32
    card_proj, wctx, bias, beta_flat, g7, bd = pl.pallas_call(
        _prep_kernel,
        out_shape=(
            jax.ShapeDtypeStruct((53, D), f32),
            jax.ShapeDtypeStruct((55, D), f32),
            jax.ShapeDtypeStruct((1, D), f32),
            jax.ShapeDtypeStruct((1, C * D), f32),
            jax.ShapeDtypeStruct((C, C * D), f32),
            jax.ShapeDtypeStruct((C * D, C), f32),
        ),
    )(card_tab, src_tab, hero_tab, actpos_tab, np_tab,
      Ws, bs.reshape(1, D), Wb, bb.reshape(1, D), Wa, ba.reshape(1, D),
      Wc, bc.reshape(1, D), gamma.reshape(1, D), beta.reshape(1, D))

    BL = B * L
    seqf = jnp.broadcast_to(
        seq_lengths.astype(f32).reshape(B, 1, 1), (B, L, 1))
    packed = jnp.concatenate([
        scalars, bets, action,
        card_ids.astype(f32),
        hero_pos.astype(f32)[:, :, None],
        acting_pos.astype(f32)[:, :, None],
        num_players.astype(f32)[:, :, None],
        seqf,
    ], axis=2).reshape(BL, _N_COLS)

    grid = (BL // EV_BLK,)
    const2 = lambda shape: pl.BlockSpec(shape, lambda i: (0, 0))
    emb, mask = pl.pallas_call(
        _main_kernel,
        grid=grid,
        in_specs=[
            pl.BlockSpec((EV_BLK, _N_COLS), lambda i: (i, 0)),
            pl.BlockSpec((B_BLK, 1), lambda i: (i, 0)),
            const2((53, D)), const2((55, D)), const2((1, D)),
            const2((1, C * D)), const2((C, C * D)), const2((C * D, C)),
        ],
        out_specs=(
            pl.BlockSpec((EV_BLK, C * D), lambda i: (i, 0)),
            pl.BlockSpec((B_BLK, L * C), lambda i: (i, 0)),
        ),
        out_shape=(
            jax.ShapeDtypeStruct((BL, C * D), f32),
            jax.ShapeDtypeStruct((B, L * C), f32),
        ),
    )(packed, seq_lengths.astype(i32).reshape(B, 1),
      card_proj, wctx, bias, beta_flat, g7, bd)
    return emb.reshape(B, L * C, D), mask
